# Initial kernel scaffold; baseline (speedup 1.0000x reference)
import jax
import jax.numpy as jnp
from jax.experimental import pallas as pl


def kernel(x, edge_index, edge_attr, batch, ligand_features, pocket_features, return_embeddings, W_msg1, b_msg1, W_edge1, b_edge1, W_self1, b_self1, W_msg2, b_msg2, W_edge2, b_edge2, W_msg3, b_msg3, W_edge3, b_edge3, W_lig, b_lig, W_poc, b_poc, W_fus, b_fus, W_out, b_out):
    raise NotImplementedError("write your pallas kernel here")


# speedup vs baseline: 7.2195x; 7.2195x over previous
import functools

import jax
import jax.numpy as jnp
from jax import lax
from jax.experimental import pallas as pl
from jax.experimental.pallas import tpu as pltpu
from jax.experimental.pallas import tpu_sc as plsc

N = 10000
E = 320000
D_IN = 128
D_EDGE = 16
H = 64
B = 64

NC = 2
NS = 16
NW = NC * NS
PER_W = E // NW
CH = 125
NCH = PER_W // CH
RPT = 624
TAIL = N - NS * RPT
TAIL_OFF = NS * RPT

BLK = 1000
NB = N // BLK


def _sc_mesh():
    return plsc.VectorSubcoreMesh(core_axis_name="c", subcore_axis_name="s")


_SC_PARAMS = pltpu.CompilerParams(use_tc_tiling_on_sc=False)


@functools.partial(
    pl.kernel,
    out_type=jax.ShapeDtypeStruct((NC, N, D_EDGE), jnp.float32),
    mesh=_sc_mesh(),
    compiler_params=_SC_PARAMS,
    scratch_types=[
        pltpu.VMEM((NCH, CH), jnp.int32),
        pltpu.VMEM((CH, D_EDGE), jnp.float32),
        pltpu.VMEM_SHARED((N, D_EDGE), jnp.float32),
        pltpu.SemaphoreType.DMA,
    ],
)
def _ea_kernel(ea_hbm, dst_hbm, zeros_hbm, out_hbm, dst_v, buf, acc, sem):
    c = lax.axis_index("c")
    s = lax.axis_index("s")
    wid = c * NS + s
    pltpu.sync_copy(zeros_hbm, acc.at[pl.ds(s * RPT, RPT)])

    @pl.when(s == NS - 1)
    def _tail_init():
        pltpu.sync_copy(zeros_hbm.at[pl.ds(0, TAIL)],
                        acc.at[pl.ds(TAIL_OFF, TAIL)])

    pltpu.sync_copy(dst_hbm.at[wid], dst_v)
    plsc.subcore_barrier()

    def body(j, carry):
        pltpu.async_copy(ea_hbm.at[wid, j], buf, sem).wait()
        pltpu.sync_copy(buf, acc.at[dst_v.at[j]], add=True)
        return carry

    lax.fori_loop(0, NCH, body, 0)
    plsc.subcore_barrier()
    pltpu.sync_copy(acc.at[pl.ds(s * RPT, RPT)],
                    out_hbm.at[c, pl.ds(s * RPT, RPT)])

    @pl.when(s == NS - 1)
    def _tail_out():
        pltpu.sync_copy(acc.at[pl.ds(TAIL_OFF, TAIL)],
                        out_hbm.at[c, pl.ds(TAIL_OFF, TAIL)])


@functools.partial(
    pl.kernel,
    out_type=jax.ShapeDtypeStruct((NC, N, H), jnp.float32),
    mesh=_sc_mesh(),
    compiler_params=_SC_PARAMS,
    scratch_types=[
        pltpu.VMEM((NCH, CH), jnp.int32),
        pltpu.VMEM((NCH, CH), jnp.int32),
        pltpu.VMEM((CH, H), jnp.float32),
        pltpu.VMEM((CH, H), jnp.float32),
        pltpu.VMEM_SHARED((N, H), jnp.float32),
        pltpu.SemaphoreType.DMA,
        pltpu.SemaphoreType.DMA,
    ],
)
def _gather_scatter_kernel(y_hbm, src_hbm, dst_hbm, zeros_hbm, out_hbm,
                           src_v, dst_v, buf0, buf1, acc, sem0, sem1):
    c = lax.axis_index("c")
    s = lax.axis_index("s")
    wid = c * NS + s
    pltpu.sync_copy(zeros_hbm, acc.at[pl.ds(s * RPT, RPT)])

    @pl.when(s == NS - 1)
    def _tail_init():
        pltpu.sync_copy(zeros_hbm.at[pl.ds(0, TAIL)],
                        acc.at[pl.ds(TAIL_OFF, TAIL)])

    pltpu.sync_copy(src_hbm.at[wid], src_v)
    pltpu.sync_copy(dst_hbm.at[wid], dst_v)
    plsc.subcore_barrier()

    def body(j, carry):
        pltpu.async_copy(y_hbm.at[src_v.at[j]], buf0, sem0).wait()
        pltpu.sync_copy(buf0, acc.at[dst_v.at[j]], add=True)
        return carry

    lax.fori_loop(0, NCH, body, 0)
    plsc.subcore_barrier()
    pltpu.sync_copy(acc.at[pl.ds(s * RPT, RPT)],
                    out_hbm.at[c, pl.ds(s * RPT, RPT)])

    @pl.when(s == NS - 1)
    def _tail_out():
        pltpu.sync_copy(acc.at[pl.ds(TAIL_OFF, TAIL)],
                        out_hbm.at[c, pl.ds(TAIL_OFF, TAIL)])


def _prep_body(x_ref, w_ref, b_ref, y1_ref, s1_ref):
    out = jnp.dot(x_ref[...], w_ref[...],
                  preferred_element_type=jnp.float32) + b_ref[...]
    y1_ref[...] = out[:, :H]
    s1_ref[...] = out[:, H:]


def _prep(x, wcat, bcat):
    return pl.pallas_call(
        _prep_body,
        grid=(NB,),
        in_specs=[
            pl.BlockSpec((BLK, D_IN), lambda i: (i, 0)),
            pl.BlockSpec((D_IN, 2 * H), lambda i: (0, 0)),
            pl.BlockSpec((1, 2 * H), lambda i: (0, 0)),
        ],
        out_specs=[
            pl.BlockSpec((BLK, H), lambda i: (i, 0)),
            pl.BlockSpec((BLK, H), lambda i: (i, 0)),
        ],
        out_shape=[
            jax.ShapeDtypeStruct((N, H), jnp.float32),
            jax.ShapeDtypeStruct((N, H), jnp.float32),
        ],
    )(x, wcat, bcat)


def _update_body(gp_ref, eap_ref, we_ref, s_ref, w_ref, b_ref, h_ref, y_ref):
    ea = eap_ref[0] + eap_ref[1]
    cterm = jnp.dot(ea, we_ref[...], preferred_element_type=jnp.float32)
    h = jnp.maximum(gp_ref[0] + gp_ref[1] + cterm + s_ref[...], 0.0)
    h_ref[...] = h
    y_ref[...] = jnp.dot(h, w_ref[...],
                         preferred_element_type=jnp.float32) + b_ref[...]


def _update(gp, eap, we, sterm, wnext, bnext):
    return pl.pallas_call(
        _update_body,
        grid=(NB,),
        in_specs=[
            pl.BlockSpec((NC, BLK, H), lambda i: (0, i, 0)),
            pl.BlockSpec((NC, BLK, D_EDGE), lambda i: (0, i, 0)),
            pl.BlockSpec((D_EDGE, H), lambda i: (0, 0)),
            pl.BlockSpec((BLK, H), lambda i: (i, 0)),
            pl.BlockSpec((H, H), lambda i: (0, 0)),
            pl.BlockSpec((1, H), lambda i: (0, 0)),
        ],
        out_specs=[
            pl.BlockSpec((BLK, H), lambda i: (i, 0)),
            pl.BlockSpec((BLK, H), lambda i: (i, 0)),
        ],
        out_shape=[
            jax.ShapeDtypeStruct((N, H), jnp.float32),
            jax.ShapeDtypeStruct((N, H), jnp.float32),
        ],
    )(gp, eap, we, sterm, wnext, bnext)


def _final_body(gp_ref, eap_ref, we_ref, s_ref, batch_ref, lig_ref, poc_ref,
                wl_ref, bl_ref, wp_ref, bp_ref, wf_ref, bf_ref, wo_ref, bo_ref,
                out_ref, acc_ref):
    i = pl.program_id(0)

    @pl.when(i == 0)
    def _init():
        acc_ref[...] = jnp.zeros_like(acc_ref)

    ea = eap_ref[0] + eap_ref[1]
    cterm = jnp.dot(ea, we_ref[...], preferred_element_type=jnp.float32)
    h3 = jnp.maximum(gp_ref[0] + gp_ref[1] + cterm + s_ref[...], 0.0)

    ids = batch_ref[0, 0, :]
    onehot = (ids[None, :] == lax.broadcasted_iota(jnp.int32, (B, BLK), 0)
              ).astype(jnp.float32)
    hcat = jnp.concatenate(
        [h3, jnp.ones((BLK, 1), jnp.float32),
         jnp.zeros((BLK, 2 * H - H - 1), jnp.float32)], axis=1)
    acc_ref[...] += jnp.dot(onehot, hcat, preferred_element_type=jnp.float32)

    @pl.when(i == NB - 1)
    def _finish():
        sums = acc_ref[:, :H]
        counts = acc_ref[:, H:H + 1]
        pooled = sums / jnp.maximum(counts, 1.0)
        lig = jnp.dot(lig_ref[...], wl_ref[...],
                      preferred_element_type=jnp.float32) + bl_ref[...]
        poc = jnp.dot(poc_ref[...], wp_ref[...],
                      preferred_element_type=jnp.float32) + bp_ref[...]
        zcat = jnp.concatenate([pooled, lig, poc], axis=1)
        z = jnp.dot(zcat, wf_ref[...],
                    preferred_element_type=jnp.float32) + bf_ref[...]
        out_ref[...] = jnp.dot(z, wo_ref[...],
                               preferred_element_type=jnp.float32) + bo_ref[...]


def _final(gp, eap, we, sterm, batch3, lig, poc, wl, bl, wp, bp, wf, bf, wo, bo):
    return pl.pallas_call(
        _final_body,
        grid=(NB,),
        in_specs=[
            pl.BlockSpec((NC, BLK, H), lambda i: (0, i, 0)),
            pl.BlockSpec((NC, BLK, D_EDGE), lambda i: (0, i, 0)),
            pl.BlockSpec((D_EDGE, H), lambda i: (0, 0)),
            pl.BlockSpec((BLK, H), lambda i: (i, 0)),
            pl.BlockSpec((1, 1, BLK), lambda i: (i, 0, 0)),
            pl.BlockSpec((B, D_IN), lambda i: (0, 0)),
            pl.BlockSpec((B, D_IN), lambda i: (0, 0)),
            pl.BlockSpec((D_IN, H), lambda i: (0, 0)),
            pl.BlockSpec((1, H), lambda i: (0, 0)),
            pl.BlockSpec((D_IN, H), lambda i: (0, 0)),
            pl.BlockSpec((1, H), lambda i: (0, 0)),
            pl.BlockSpec((3 * H, H), lambda i: (0, 0)),
            pl.BlockSpec((1, H), lambda i: (0, 0)),
            pl.BlockSpec((H, 1), lambda i: (0, 0)),
            pl.BlockSpec((1, 1), lambda i: (0, 0)),
        ],
        out_specs=pl.BlockSpec((B, 1), lambda i: (0, 0)),
        out_shape=jax.ShapeDtypeStruct((B, 1), jnp.float32),
        scratch_shapes=[pltpu.VMEM((B, 2 * H), jnp.float32)],
    )(gp, eap, we, sterm, batch3, lig, poc, wl, bl, wp, bp, wf, bf, wo, bo)


def kernel(x, edge_index, edge_attr, batch, ligand_features, pocket_features,
           return_embeddings, W_msg1, b_msg1, W_edge1, b_edge1, W_self1,
           b_self1, W_msg2, b_msg2, W_edge2, b_edge2, W_msg3, b_msg3, W_edge3,
           b_edge3, W_lig, b_lig, W_poc, b_poc, W_fus, b_fus, W_out, b_out):
    src_r = edge_index[0].reshape(NW, NCH, CH)
    dst_r = edge_index[1].reshape(NW, NCH, CH)
    ea_r = edge_attr.reshape(NW, NCH, CH, D_EDGE)
    batch3 = batch.reshape(NB, 1, BLK)
    z16 = jnp.zeros((RPT, D_EDGE), jnp.float32)
    z64 = jnp.zeros((RPT, H), jnp.float32)

    eap = _ea_kernel(ea_r, dst_r, z16)

    wcat = jnp.concatenate([W_msg1, W_self1], axis=1)
    bcat = jnp.concatenate([b_msg1 + b_edge1, b_self1]).reshape(1, 2 * H)
    y1, s1 = _prep(x, wcat, bcat)

    g1 = _gather_scatter_kernel(y1, src_r, dst_r, z64)
    h1, y2 = _update(g1, eap, W_edge1, s1, W_msg2,
                     (b_msg2 + b_edge2).reshape(1, H))
    g2 = _gather_scatter_kernel(y2, src_r, dst_r, z64)
    h2, y3 = _update(g2, eap, W_edge2, h1, W_msg3,
                     (b_msg3 + b_edge3).reshape(1, H))
    g3 = _gather_scatter_kernel(y3, src_r, dst_r, z64)
    out = _final(g3, eap, W_edge3, h2, batch3, ligand_features,
                 pocket_features, W_lig, b_lig.reshape(1, H), W_poc,
                 b_poc.reshape(1, H), W_fus, b_fus.reshape(1, H), W_out,
                 b_out.reshape(1, 1))

    gate = jnp.asarray(return_embeddings, jnp.float32)
    return out * (1.0 - gate)


# speedup vs baseline: 8.3418x; 1.1554x over previous
import functools

import jax
import jax.numpy as jnp
from jax import lax
from jax.experimental import pallas as pl
from jax.experimental.pallas import tpu as pltpu
from jax.experimental.pallas import tpu_sc as plsc

N = 10000
E = 320000
D_IN = 128
D_EDGE = 16
H = 64
B = 64

NC = 2
NS = 16
NW = NC * NS
PER_W = E // NW
CH = 125
NCH = PER_W // CH
RPT = 624
TAIL = N - NS * RPT
TAIL_OFF = NS * RPT

BLK = 1000
NB = N // BLK


def _sc_mesh():
    return plsc.VectorSubcoreMesh(core_axis_name="c", subcore_axis_name="s")


_SC_PARAMS = pltpu.CompilerParams(use_tc_tiling_on_sc=False)


@functools.partial(
    pl.kernel,
    out_type=jax.ShapeDtypeStruct((NC, N, D_EDGE), jnp.float32),
    mesh=_sc_mesh(),
    compiler_params=_SC_PARAMS,
    scratch_types=[
        pltpu.VMEM((NCH, CH), jnp.int32),
        pltpu.VMEM((CH, D_EDGE), jnp.float32),
        pltpu.VMEM((CH, D_EDGE), jnp.float32),
        pltpu.VMEM_SHARED((N, D_EDGE), jnp.float32),
        pltpu.SemaphoreType.DMA,
        pltpu.SemaphoreType.DMA,
    ],
)
def _ea_kernel(ea_hbm, dst_hbm, zeros_hbm, out_hbm, dst_v, buf0, buf1, acc,
               sem0, sem1):
    c = lax.axis_index("c")
    s = lax.axis_index("s")
    wid = c * NS + s
    pltpu.sync_copy(zeros_hbm, acc.at[pl.ds(s * RPT, RPT)])

    @pl.when(s == NS - 1)
    def _tail_init():
        pltpu.sync_copy(zeros_hbm.at[pl.ds(0, TAIL)],
                        acc.at[pl.ds(TAIL_OFF, TAIL)])

    pltpu.sync_copy(dst_hbm.at[wid], dst_v)
    plsc.subcore_barrier()

    pltpu.async_copy(ea_hbm.at[wid, 0], buf0, sem0)

    def body(jj, carry):
        j0 = 2 * jj
        pltpu.make_async_copy(ea_hbm.at[wid, j0], buf0, sem0).wait()
        pltpu.async_copy(ea_hbm.at[wid, j0 + 1], buf1, sem1)
        pltpu.sync_copy(buf0, acc.at[dst_v.at[j0]], add=True)

        pltpu.make_async_copy(ea_hbm.at[wid, j0 + 1], buf1, sem1).wait()

        @pl.when(j0 + 2 < NCH)
        def _pre():
            pltpu.async_copy(ea_hbm.at[wid, j0 + 2], buf0, sem0)

        pltpu.sync_copy(buf1, acc.at[dst_v.at[j0 + 1]], add=True)
        return carry

    lax.fori_loop(0, NCH // 2, body, 0)
    plsc.subcore_barrier()
    pltpu.sync_copy(acc.at[pl.ds(s * RPT, RPT)],
                    out_hbm.at[c, pl.ds(s * RPT, RPT)])

    @pl.when(s == NS - 1)
    def _tail_out():
        pltpu.sync_copy(acc.at[pl.ds(TAIL_OFF, TAIL)],
                        out_hbm.at[c, pl.ds(TAIL_OFF, TAIL)])


@functools.partial(
    pl.kernel,
    out_type=jax.ShapeDtypeStruct((NC, N, H), jnp.float32),
    mesh=_sc_mesh(),
    compiler_params=_SC_PARAMS,
    scratch_types=[
        pltpu.VMEM((NCH, CH), jnp.int32),
        pltpu.VMEM((NCH, CH), jnp.int32),
        pltpu.VMEM((CH, H), jnp.float32),
        pltpu.VMEM((CH, H), jnp.float32),
        pltpu.VMEM_SHARED((N, H), jnp.float32),
        pltpu.SemaphoreType.DMA,
        pltpu.SemaphoreType.DMA,
    ],
)
def _gather_scatter_kernel(y_hbm, src_hbm, dst_hbm, zeros_hbm, out_hbm,
                           src_v, dst_v, buf0, buf1, acc, sem0, sem1):
    c = lax.axis_index("c")
    s = lax.axis_index("s")
    wid = c * NS + s
    pltpu.sync_copy(zeros_hbm, acc.at[pl.ds(s * RPT, RPT)])

    @pl.when(s == NS - 1)
    def _tail_init():
        pltpu.sync_copy(zeros_hbm.at[pl.ds(0, TAIL)],
                        acc.at[pl.ds(TAIL_OFF, TAIL)])

    pltpu.sync_copy(src_hbm.at[wid], src_v)
    pltpu.sync_copy(dst_hbm.at[wid], dst_v)
    plsc.subcore_barrier()

    pltpu.async_copy(y_hbm.at[src_v.at[0]], buf0, sem0)

    def body(jj, carry):
        j0 = 2 * jj

        pltpu.make_async_copy(y_hbm.at[src_v.at[j0]], buf0, sem0).wait()
        pltpu.async_copy(y_hbm.at[src_v.at[j0 + 1]], buf1, sem1)
        pltpu.sync_copy(buf0, acc.at[dst_v.at[j0]], add=True)

        pltpu.make_async_copy(y_hbm.at[src_v.at[j0 + 1]], buf1, sem1).wait()

        @pl.when(j0 + 2 < NCH)
        def _pre():
            pltpu.async_copy(y_hbm.at[src_v.at[j0 + 2]], buf0, sem0)

        pltpu.sync_copy(buf1, acc.at[dst_v.at[j0 + 1]], add=True)
        return carry

    lax.fori_loop(0, NCH // 2, body, 0)
    plsc.subcore_barrier()
    pltpu.sync_copy(acc.at[pl.ds(s * RPT, RPT)],
                    out_hbm.at[c, pl.ds(s * RPT, RPT)])

    @pl.when(s == NS - 1)
    def _tail_out():
        pltpu.sync_copy(acc.at[pl.ds(TAIL_OFF, TAIL)],
                        out_hbm.at[c, pl.ds(TAIL_OFF, TAIL)])


def _prep_body(x_ref, w_ref, b_ref, y1_ref, s1_ref):
    out = jnp.dot(x_ref[...], w_ref[...],
                  preferred_element_type=jnp.float32) + b_ref[...]
    y1_ref[...] = out[:, :H]
    s1_ref[...] = out[:, H:]


def _prep(x, wcat, bcat):
    return pl.pallas_call(
        _prep_body,
        grid=(NB,),
        in_specs=[
            pl.BlockSpec((BLK, D_IN), lambda i: (i, 0)),
            pl.BlockSpec((D_IN, 2 * H), lambda i: (0, 0)),
            pl.BlockSpec((1, 2 * H), lambda i: (0, 0)),
        ],
        out_specs=[
            pl.BlockSpec((BLK, H), lambda i: (i, 0)),
            pl.BlockSpec((BLK, H), lambda i: (i, 0)),
        ],
        out_shape=[
            jax.ShapeDtypeStruct((N, H), jnp.float32),
            jax.ShapeDtypeStruct((N, H), jnp.float32),
        ],
    )(x, wcat, bcat)


def _update_body(gp_ref, eap_ref, we_ref, s_ref, w_ref, b_ref, h_ref, y_ref):
    ea = eap_ref[0] + eap_ref[1]
    cterm = jnp.dot(ea, we_ref[...], preferred_element_type=jnp.float32)
    h = jnp.maximum(gp_ref[0] + gp_ref[1] + cterm + s_ref[...], 0.0)
    h_ref[...] = h
    y_ref[...] = jnp.dot(h, w_ref[...],
                         preferred_element_type=jnp.float32) + b_ref[...]


def _update(gp, eap, we, sterm, wnext, bnext):
    return pl.pallas_call(
        _update_body,
        grid=(NB,),
        in_specs=[
            pl.BlockSpec((NC, BLK, H), lambda i: (0, i, 0)),
            pl.BlockSpec((NC, BLK, D_EDGE), lambda i: (0, i, 0)),
            pl.BlockSpec((D_EDGE, H), lambda i: (0, 0)),
            pl.BlockSpec((BLK, H), lambda i: (i, 0)),
            pl.BlockSpec((H, H), lambda i: (0, 0)),
            pl.BlockSpec((1, H), lambda i: (0, 0)),
        ],
        out_specs=[
            pl.BlockSpec((BLK, H), lambda i: (i, 0)),
            pl.BlockSpec((BLK, H), lambda i: (i, 0)),
        ],
        out_shape=[
            jax.ShapeDtypeStruct((N, H), jnp.float32),
            jax.ShapeDtypeStruct((N, H), jnp.float32),
        ],
    )(gp, eap, we, sterm, wnext, bnext)


def _final_body(gp_ref, eap_ref, we_ref, s_ref, batch_ref, lig_ref, poc_ref,
                wl_ref, bl_ref, wp_ref, bp_ref, wf_ref, bf_ref, wo_ref, bo_ref,
                out_ref, acc_ref):
    i = pl.program_id(0)

    @pl.when(i == 0)
    def _init():
        acc_ref[...] = jnp.zeros_like(acc_ref)

    ea = eap_ref[0] + eap_ref[1]
    cterm = jnp.dot(ea, we_ref[...], preferred_element_type=jnp.float32)
    h3 = jnp.maximum(gp_ref[0] + gp_ref[1] + cterm + s_ref[...], 0.0)

    ids = batch_ref[0, 0, :]
    onehot = (ids[None, :] == lax.broadcasted_iota(jnp.int32, (B, BLK), 0)
              ).astype(jnp.float32)
    hcat = jnp.concatenate(
        [h3, jnp.ones((BLK, 1), jnp.float32),
         jnp.zeros((BLK, 2 * H - H - 1), jnp.float32)], axis=1)
    acc_ref[...] += jnp.dot(onehot, hcat, preferred_element_type=jnp.float32)

    @pl.when(i == NB - 1)
    def _finish():
        sums = acc_ref[:, :H]
        counts = acc_ref[:, H:H + 1]
        pooled = sums / jnp.maximum(counts, 1.0)
        lig = jnp.dot(lig_ref[...], wl_ref[...],
                      preferred_element_type=jnp.float32) + bl_ref[...]
        poc = jnp.dot(poc_ref[...], wp_ref[...],
                      preferred_element_type=jnp.float32) + bp_ref[...]
        zcat = jnp.concatenate([pooled, lig, poc], axis=1)
        z = jnp.dot(zcat, wf_ref[...],
                    preferred_element_type=jnp.float32) + bf_ref[...]
        out_ref[...] = jnp.dot(z, wo_ref[...],
                               preferred_element_type=jnp.float32) + bo_ref[...]


def _final(gp, eap, we, sterm, batch3, lig, poc, wl, bl, wp, bp, wf, bf, wo, bo):
    return pl.pallas_call(
        _final_body,
        grid=(NB,),
        in_specs=[
            pl.BlockSpec((NC, BLK, H), lambda i: (0, i, 0)),
            pl.BlockSpec((NC, BLK, D_EDGE), lambda i: (0, i, 0)),
            pl.BlockSpec((D_EDGE, H), lambda i: (0, 0)),
            pl.BlockSpec((BLK, H), lambda i: (i, 0)),
            pl.BlockSpec((1, 1, BLK), lambda i: (i, 0, 0)),
            pl.BlockSpec((B, D_IN), lambda i: (0, 0)),
            pl.BlockSpec((B, D_IN), lambda i: (0, 0)),
            pl.BlockSpec((D_IN, H), lambda i: (0, 0)),
            pl.BlockSpec((1, H), lambda i: (0, 0)),
            pl.BlockSpec((D_IN, H), lambda i: (0, 0)),
            pl.BlockSpec((1, H), lambda i: (0, 0)),
            pl.BlockSpec((3 * H, H), lambda i: (0, 0)),
            pl.BlockSpec((1, H), lambda i: (0, 0)),
            pl.BlockSpec((H, 1), lambda i: (0, 0)),
            pl.BlockSpec((1, 1), lambda i: (0, 0)),
        ],
        out_specs=pl.BlockSpec((B, 1), lambda i: (0, 0)),
        out_shape=jax.ShapeDtypeStruct((B, 1), jnp.float32),
        scratch_shapes=[pltpu.VMEM((B, 2 * H), jnp.float32)],
    )(gp, eap, we, sterm, batch3, lig, poc, wl, bl, wp, bp, wf, bf, wo, bo)


def kernel(x, edge_index, edge_attr, batch, ligand_features, pocket_features,
           return_embeddings, W_msg1, b_msg1, W_edge1, b_edge1, W_self1,
           b_self1, W_msg2, b_msg2, W_edge2, b_edge2, W_msg3, b_msg3, W_edge3,
           b_edge3, W_lig, b_lig, W_poc, b_poc, W_fus, b_fus, W_out, b_out):
    src_r = edge_index[0].reshape(NW, NCH, CH)
    dst_r = edge_index[1].reshape(NW, NCH, CH)
    ea_r = edge_attr.reshape(NW, NCH, CH, D_EDGE)
    batch3 = batch.reshape(NB, 1, BLK)
    z16 = jnp.zeros((RPT, D_EDGE), jnp.float32)
    z64 = jnp.zeros((RPT, H), jnp.float32)

    eap = _ea_kernel(ea_r, dst_r, z16)

    wcat = jnp.concatenate([W_msg1, W_self1], axis=1)
    bcat = jnp.concatenate([b_msg1 + b_edge1, b_self1]).reshape(1, 2 * H)
    y1, s1 = _prep(x, wcat, bcat)

    g1 = _gather_scatter_kernel(y1, src_r, dst_r, z64)
    h1, y2 = _update(g1, eap, W_edge1, s1, W_msg2,
                     (b_msg2 + b_edge2).reshape(1, H))
    g2 = _gather_scatter_kernel(y2, src_r, dst_r, z64)
    h2, y3 = _update(g2, eap, W_edge2, h1, W_msg3,
                     (b_msg3 + b_edge3).reshape(1, H))
    g3 = _gather_scatter_kernel(y3, src_r, dst_r, z64)
    out = _final(g3, eap, W_edge3, h2, batch3, ligand_features,
                 pocket_features, W_lig, b_lig.reshape(1, H), W_poc,
                 b_poc.reshape(1, H), W_fus, b_fus.reshape(1, H), W_out,
                 b_out.reshape(1, 1))

    gate = jnp.asarray(return_embeddings, jnp.float32)
    return out * (1.0 - gate)


# speedup vs baseline: 9.7776x; 1.1721x over previous
import functools

import jax
import jax.numpy as jnp
from jax import lax
from jax.experimental import pallas as pl
from jax.experimental.pallas import tpu as pltpu
from jax.experimental.pallas import tpu_sc as plsc

N = 10000
E = 320000
D_IN = 128
D_EDGE = 16
H = 64
B = 64

NC = 2
NS = 16
NW = NC * NS
PER_W = E // NW
CH = 125
NCH = PER_W // CH
RPT = 624
TAIL = N - NS * RPT
TAIL_OFF = NS * RPT

BLK = 1000
NB = N // BLK


def _sc_mesh():
    return plsc.VectorSubcoreMesh(core_axis_name="c", subcore_axis_name="s")


_SC_PARAMS = pltpu.CompilerParams(use_tc_tiling_on_sc=False)


@functools.partial(
    pl.kernel,
    out_type=jax.ShapeDtypeStruct((NC, N, D_EDGE), jnp.float32),
    mesh=_sc_mesh(),
    compiler_params=_SC_PARAMS,
    scratch_types=[
        pltpu.VMEM((NCH, CH), jnp.int32),
        pltpu.VMEM((CH, D_EDGE), jnp.float32),
        pltpu.VMEM((CH, D_EDGE), jnp.float32),
        pltpu.VMEM_SHARED((N, D_EDGE), jnp.float32),
        pltpu.SemaphoreType.DMA,
        pltpu.SemaphoreType.DMA,
    ],
)
def _ea_kernel(ea_hbm, dst_hbm, zeros_hbm, out_hbm, dst_v, buf0, buf1, acc,
               sem0, sem1):
    c = lax.axis_index("c")
    s = lax.axis_index("s")
    wid = c * NS + s
    pltpu.sync_copy(zeros_hbm, acc.at[pl.ds(s * RPT, RPT)])

    @pl.when(s == NS - 1)
    def _tail_init():
        pltpu.sync_copy(zeros_hbm.at[pl.ds(0, TAIL)],
                        acc.at[pl.ds(TAIL_OFF, TAIL)])

    pltpu.sync_copy(dst_hbm.at[wid], dst_v)
    plsc.subcore_barrier()

    pltpu.async_copy(ea_hbm.at[wid, 0], buf0, sem0)

    def body(jj, carry):
        j0 = 2 * jj
        pltpu.make_async_copy(ea_hbm.at[wid, j0], buf0, sem0).wait()
        pltpu.async_copy(ea_hbm.at[wid, j0 + 1], buf1, sem1)
        pltpu.sync_copy(buf0, acc.at[dst_v.at[j0]], add=True)

        pltpu.make_async_copy(ea_hbm.at[wid, j0 + 1], buf1, sem1).wait()

        @pl.when(j0 + 2 < NCH)
        def _pre():
            pltpu.async_copy(ea_hbm.at[wid, j0 + 2], buf0, sem0)

        pltpu.sync_copy(buf1, acc.at[dst_v.at[j0 + 1]], add=True)
        return carry

    lax.fori_loop(0, NCH // 2, body, 0)
    plsc.subcore_barrier()
    pltpu.sync_copy(acc.at[pl.ds(s * RPT, RPT)],
                    out_hbm.at[c, pl.ds(s * RPT, RPT)])

    @pl.when(s == NS - 1)
    def _tail_out():
        pltpu.sync_copy(acc.at[pl.ds(TAIL_OFF, TAIL)],
                        out_hbm.at[c, pl.ds(TAIL_OFF, TAIL)])


@functools.partial(
    pl.kernel,
    out_type=jax.ShapeDtypeStruct((NC, N, H), jnp.float32),
    mesh=_sc_mesh(),
    compiler_params=_SC_PARAMS,
    scratch_types=[
        pltpu.VMEM((NCH, CH), jnp.int32),
        pltpu.VMEM((NCH, CH), jnp.int32),
        pltpu.VMEM((CH, H), jnp.float32),
        pltpu.VMEM((CH, H), jnp.float32),
        pltpu.VMEM((CH, H), jnp.float32),
        pltpu.VMEM((CH, H), jnp.float32),
        pltpu.VMEM_SHARED((N, H), jnp.float32),
        [pltpu.SemaphoreType.DMA] * 4,
        [pltpu.SemaphoreType.DMA] * 4,
    ],
)
def _gather_scatter_kernel(y_hbm, src_hbm, dst_hbm, zeros_hbm, out_hbm,
                           src_v, dst_v, buf0, buf1, buf2, buf3, acc,
                           gsem, ssem):
    c = lax.axis_index("c")
    s = lax.axis_index("s")
    wid = c * NS + s
    pltpu.sync_copy(zeros_hbm, acc.at[pl.ds(s * RPT, RPT)])

    @pl.when(s == NS - 1)
    def _tail_init():
        pltpu.sync_copy(zeros_hbm.at[pl.ds(0, TAIL)],
                        acc.at[pl.ds(TAIL_OFF, TAIL)])

    pltpu.sync_copy(src_hbm.at[wid], src_v)
    pltpu.sync_copy(dst_hbm.at[wid], dst_v)
    plsc.subcore_barrier()

    bufs = (buf0, buf1, buf2, buf3)
    for p in range(3):
        pltpu.async_copy(y_hbm.at[src_v.at[p]], bufs[p], gsem[p])

    def body(jj, carry):
        for p in range(4):
            j = 4 * jj + p
            q = (p + 3) % 4
            pltpu.make_async_copy(y_hbm.at[src_v.at[j]], bufs[p],
                                  gsem[p]).wait()
            pltpu.async_copy(bufs[p], acc.at[dst_v.at[j]], ssem[p], add=True)

            @pl.when((j >= 1) & (j + 3 < NCH))
            def _drain(q=q, j=j):
                pltpu.make_async_copy(bufs[q], acc.at[dst_v.at[j]],
                                      ssem[q]).wait()

            @pl.when(j + 3 < NCH)
            def _prefetch(q=q, j=j):
                pltpu.async_copy(y_hbm.at[src_v.at[j + 3]], bufs[q], gsem[q])
        return carry

    lax.fori_loop(0, NCH // 4, body, 0)
    for p in range(4):
        pltpu.make_async_copy(bufs[p], acc.at[dst_v.at[0]], ssem[p]).wait()
    plsc.subcore_barrier()
    pltpu.sync_copy(acc.at[pl.ds(s * RPT, RPT)],
                    out_hbm.at[c, pl.ds(s * RPT, RPT)])

    @pl.when(s == NS - 1)
    def _tail_out():
        pltpu.sync_copy(acc.at[pl.ds(TAIL_OFF, TAIL)],
                        out_hbm.at[c, pl.ds(TAIL_OFF, TAIL)])


def _prep_body(x_ref, w_ref, b_ref, y1_ref, s1_ref):
    out = jnp.dot(x_ref[...], w_ref[...],
                  preferred_element_type=jnp.float32) + b_ref[...]
    y1_ref[...] = out[:, :H]
    s1_ref[...] = out[:, H:]


def _prep(x, wcat, bcat):
    return pl.pallas_call(
        _prep_body,
        grid=(NB,),
        in_specs=[
            pl.BlockSpec((BLK, D_IN), lambda i: (i, 0)),
            pl.BlockSpec((D_IN, 2 * H), lambda i: (0, 0)),
            pl.BlockSpec((1, 2 * H), lambda i: (0, 0)),
        ],
        out_specs=[
            pl.BlockSpec((BLK, H), lambda i: (i, 0)),
            pl.BlockSpec((BLK, H), lambda i: (i, 0)),
        ],
        out_shape=[
            jax.ShapeDtypeStruct((N, H), jnp.float32),
            jax.ShapeDtypeStruct((N, H), jnp.float32),
        ],
    )(x, wcat, bcat)


def _update_body(gp_ref, eap_ref, we_ref, s_ref, w_ref, b_ref, h_ref, y_ref):
    ea = eap_ref[0] + eap_ref[1]
    cterm = jnp.dot(ea, we_ref[...], preferred_element_type=jnp.float32)
    h = jnp.maximum(gp_ref[0] + gp_ref[1] + cterm + s_ref[...], 0.0)
    h_ref[...] = h
    y_ref[...] = jnp.dot(h, w_ref[...],
                         preferred_element_type=jnp.float32) + b_ref[...]


def _update(gp, eap, we, sterm, wnext, bnext):
    return pl.pallas_call(
        _update_body,
        grid=(NB,),
        in_specs=[
            pl.BlockSpec((NC, BLK, H), lambda i: (0, i, 0)),
            pl.BlockSpec((NC, BLK, D_EDGE), lambda i: (0, i, 0)),
            pl.BlockSpec((D_EDGE, H), lambda i: (0, 0)),
            pl.BlockSpec((BLK, H), lambda i: (i, 0)),
            pl.BlockSpec((H, H), lambda i: (0, 0)),
            pl.BlockSpec((1, H), lambda i: (0, 0)),
        ],
        out_specs=[
            pl.BlockSpec((BLK, H), lambda i: (i, 0)),
            pl.BlockSpec((BLK, H), lambda i: (i, 0)),
        ],
        out_shape=[
            jax.ShapeDtypeStruct((N, H), jnp.float32),
            jax.ShapeDtypeStruct((N, H), jnp.float32),
        ],
    )(gp, eap, we, sterm, wnext, bnext)


def _final_body(gp_ref, eap_ref, we_ref, s_ref, batch_ref, lig_ref, poc_ref,
                wl_ref, bl_ref, wp_ref, bp_ref, wf_ref, bf_ref, wo_ref, bo_ref,
                out_ref, acc_ref):
    i = pl.program_id(0)

    @pl.when(i == 0)
    def _init():
        acc_ref[...] = jnp.zeros_like(acc_ref)

    ea = eap_ref[0] + eap_ref[1]
    cterm = jnp.dot(ea, we_ref[...], preferred_element_type=jnp.float32)
    h3 = jnp.maximum(gp_ref[0] + gp_ref[1] + cterm + s_ref[...], 0.0)

    ids = batch_ref[0, 0, :]
    onehot = (ids[None, :] == lax.broadcasted_iota(jnp.int32, (B, BLK), 0)
              ).astype(jnp.float32)
    hcat = jnp.concatenate(
        [h3, jnp.ones((BLK, 1), jnp.float32),
         jnp.zeros((BLK, 2 * H - H - 1), jnp.float32)], axis=1)
    acc_ref[...] += jnp.dot(onehot, hcat, preferred_element_type=jnp.float32)

    @pl.when(i == NB - 1)
    def _finish():
        sums = acc_ref[:, :H]
        counts = acc_ref[:, H:H + 1]
        pooled = sums / jnp.maximum(counts, 1.0)
        lig = jnp.dot(lig_ref[...], wl_ref[...],
                      preferred_element_type=jnp.float32) + bl_ref[...]
        poc = jnp.dot(poc_ref[...], wp_ref[...],
                      preferred_element_type=jnp.float32) + bp_ref[...]
        zcat = jnp.concatenate([pooled, lig, poc], axis=1)
        z = jnp.dot(zcat, wf_ref[...],
                    preferred_element_type=jnp.float32) + bf_ref[...]
        out_ref[...] = jnp.dot(z, wo_ref[...],
                               preferred_element_type=jnp.float32) + bo_ref[...]


def _final(gp, eap, we, sterm, batch3, lig, poc, wl, bl, wp, bp, wf, bf, wo, bo):
    return pl.pallas_call(
        _final_body,
        grid=(NB,),
        in_specs=[
            pl.BlockSpec((NC, BLK, H), lambda i: (0, i, 0)),
            pl.BlockSpec((NC, BLK, D_EDGE), lambda i: (0, i, 0)),
            pl.BlockSpec((D_EDGE, H), lambda i: (0, 0)),
            pl.BlockSpec((BLK, H), lambda i: (i, 0)),
            pl.BlockSpec((1, 1, BLK), lambda i: (i, 0, 0)),
            pl.BlockSpec((B, D_IN), lambda i: (0, 0)),
            pl.BlockSpec((B, D_IN), lambda i: (0, 0)),
            pl.BlockSpec((D_IN, H), lambda i: (0, 0)),
            pl.BlockSpec((1, H), lambda i: (0, 0)),
            pl.BlockSpec((D_IN, H), lambda i: (0, 0)),
            pl.BlockSpec((1, H), lambda i: (0, 0)),
            pl.BlockSpec((3 * H, H), lambda i: (0, 0)),
            pl.BlockSpec((1, H), lambda i: (0, 0)),
            pl.BlockSpec((H, 1), lambda i: (0, 0)),
            pl.BlockSpec((1, 1), lambda i: (0, 0)),
        ],
        out_specs=pl.BlockSpec((B, 1), lambda i: (0, 0)),
        out_shape=jax.ShapeDtypeStruct((B, 1), jnp.float32),
        scratch_shapes=[pltpu.VMEM((B, 2 * H), jnp.float32)],
    )(gp, eap, we, sterm, batch3, lig, poc, wl, bl, wp, bp, wf, bf, wo, bo)


def kernel(x, edge_index, edge_attr, batch, ligand_features, pocket_features,
           return_embeddings, W_msg1, b_msg1, W_edge1, b_edge1, W_self1,
           b_self1, W_msg2, b_msg2, W_edge2, b_edge2, W_msg3, b_msg3, W_edge3,
           b_edge3, W_lig, b_lig, W_poc, b_poc, W_fus, b_fus, W_out, b_out):
    src_r = edge_index[0].reshape(NW, NCH, CH)
    dst_r = edge_index[1].reshape(NW, NCH, CH)
    ea_r = edge_attr.reshape(NW, NCH, CH, D_EDGE)
    batch3 = batch.reshape(NB, 1, BLK)
    z16 = jnp.zeros((RPT, D_EDGE), jnp.float32)
    z64 = jnp.zeros((RPT, H), jnp.float32)

    eap = _ea_kernel(ea_r, dst_r, z16)

    wcat = jnp.concatenate([W_msg1, W_self1], axis=1)
    bcat = jnp.concatenate([b_msg1 + b_edge1, b_self1]).reshape(1, 2 * H)
    y1, s1 = _prep(x, wcat, bcat)

    g1 = _gather_scatter_kernel(y1, src_r, dst_r, z64)
    h1, y2 = _update(g1, eap, W_edge1, s1, W_msg2,
                     (b_msg2 + b_edge2).reshape(1, H))
    g2 = _gather_scatter_kernel(y2, src_r, dst_r, z64)
    h2, y3 = _update(g2, eap, W_edge2, h1, W_msg3,
                     (b_msg3 + b_edge3).reshape(1, H))
    g3 = _gather_scatter_kernel(y3, src_r, dst_r, z64)
    out = _final(g3, eap, W_edge3, h2, batch3, ligand_features,
                 pocket_features, W_lig, b_lig.reshape(1, H), W_poc,
                 b_poc.reshape(1, H), W_fus, b_fus.reshape(1, H), W_out,
                 b_out.reshape(1, 1))

    gate = jnp.asarray(return_embeddings, jnp.float32)
    return out * (1.0 - gate)


# speedup vs baseline: 10.4885x; 1.0727x over previous
import functools

import jax
import jax.numpy as jnp
from jax import lax
from jax.experimental import pallas as pl
from jax.experimental.pallas import tpu as pltpu
from jax.experimental.pallas import tpu_sc as plsc

N = 10000
E = 320000
D_IN = 128
D_EDGE = 16
H = 64
B = 64

NC = 2
NS = 16
NW = NC * NS
PER_W = E // NW
CH = 125
NCH = PER_W // CH
RPT = 624
TAIL = N - NS * RPT
TAIL_OFF = NS * RPT

BLK = 1000
NB = N // BLK


def _sc_mesh():
    return plsc.VectorSubcoreMesh(core_axis_name="c", subcore_axis_name="s")


_SC_PARAMS = pltpu.CompilerParams(use_tc_tiling_on_sc=False)


@functools.partial(
    pl.kernel,
    out_type=jax.ShapeDtypeStruct((NC, N, D_EDGE), jnp.float32),
    mesh=_sc_mesh(),
    compiler_params=_SC_PARAMS,
    scratch_types=[
        pltpu.VMEM((NCH, CH), jnp.int32),
        pltpu.VMEM((CH, D_EDGE), jnp.float32),
        pltpu.VMEM((CH, D_EDGE), jnp.float32),
        pltpu.VMEM((CH, D_EDGE), jnp.float32),
        pltpu.VMEM((CH, D_EDGE), jnp.float32),
        pltpu.VMEM_SHARED((N, D_EDGE), jnp.float32),
        [pltpu.SemaphoreType.DMA] * 4,
        [pltpu.SemaphoreType.DMA] * 4,
    ],
)
def _ea_kernel(ea_hbm, dst_hbm, zeros_hbm, out_hbm, dst_v, buf0, buf1, buf2,
               buf3, acc, gsem, ssem):
    c = lax.axis_index("c")
    s = lax.axis_index("s")
    wid = c * NS + s
    pltpu.sync_copy(zeros_hbm, acc.at[pl.ds(s * RPT, RPT)])

    @pl.when(s == NS - 1)
    def _tail_init():
        pltpu.sync_copy(zeros_hbm.at[pl.ds(0, TAIL)],
                        acc.at[pl.ds(TAIL_OFF, TAIL)])

    pltpu.sync_copy(dst_hbm.at[wid], dst_v)
    plsc.subcore_barrier()

    bufs = (buf0, buf1, buf2, buf3)
    for p in range(3):
        pltpu.async_copy(ea_hbm.at[wid, p], bufs[p], gsem[p])

    def body(jj, carry):
        for p in range(4):
            j = 4 * jj + p
            q = (p + 3) % 4
            pltpu.make_async_copy(ea_hbm.at[wid, j], bufs[p], gsem[p]).wait()
            pltpu.async_copy(bufs[p], acc.at[dst_v.at[j]], ssem[p], add=True)

            @pl.when((j >= 1) & (j + 3 < NCH))
            def _drain(q=q, j=j):
                pltpu.make_async_copy(bufs[q], acc.at[dst_v.at[j]],
                                      ssem[q]).wait()

            @pl.when(j + 3 < NCH)
            def _prefetch(q=q, j=j):
                pltpu.async_copy(ea_hbm.at[wid, j + 3], bufs[q], gsem[q])
        return carry

    lax.fori_loop(0, NCH // 4, body, 0)
    for p in range(4):
        pltpu.make_async_copy(bufs[p], acc.at[dst_v.at[0]], ssem[p]).wait()
    plsc.subcore_barrier()
    pltpu.sync_copy(acc.at[pl.ds(s * RPT, RPT)],
                    out_hbm.at[c, pl.ds(s * RPT, RPT)])

    @pl.when(s == NS - 1)
    def _tail_out():
        pltpu.sync_copy(acc.at[pl.ds(TAIL_OFF, TAIL)],
                        out_hbm.at[c, pl.ds(TAIL_OFF, TAIL)])


@functools.partial(
    pl.kernel,
    out_type=jax.ShapeDtypeStruct((NC, N, H), jnp.float32),
    mesh=_sc_mesh(),
    compiler_params=_SC_PARAMS,
    scratch_types=[
        pltpu.VMEM((NCH, CH), jnp.int32),
        pltpu.VMEM((NCH, CH), jnp.int32),
        pltpu.VMEM((CH, H), jnp.float32),
        pltpu.VMEM((CH, H), jnp.float32),
        pltpu.VMEM((CH, H), jnp.float32),
        pltpu.VMEM((CH, H), jnp.float32),
        pltpu.VMEM_SHARED((N, H), jnp.float32),
        [pltpu.SemaphoreType.DMA] * 4,
        [pltpu.SemaphoreType.DMA] * 4,
    ],
)
def _gather_scatter_kernel(y_hbm, src_hbm, dst_hbm, zeros_hbm, out_hbm,
                           src_v, dst_v, buf0, buf1, buf2, buf3, acc,
                           gsem, ssem):
    c = lax.axis_index("c")
    s = lax.axis_index("s")
    wid = c * NS + s
    pltpu.sync_copy(zeros_hbm, acc.at[pl.ds(s * RPT, RPT)])

    @pl.when(s == NS - 1)
    def _tail_init():
        pltpu.sync_copy(zeros_hbm.at[pl.ds(0, TAIL)],
                        acc.at[pl.ds(TAIL_OFF, TAIL)])

    pltpu.sync_copy(src_hbm.at[wid], src_v)
    pltpu.sync_copy(dst_hbm.at[wid], dst_v)
    plsc.subcore_barrier()

    bufs = (buf0, buf1, buf2, buf3)
    for p in range(3):
        pltpu.async_copy(y_hbm.at[src_v.at[p]], bufs[p], gsem[p])

    def body(jj, carry):
        for p in range(4):
            j = 4 * jj + p
            q = (p + 3) % 4
            pltpu.make_async_copy(y_hbm.at[src_v.at[j]], bufs[p],
                                  gsem[p]).wait()
            pltpu.async_copy(bufs[p], acc.at[dst_v.at[j]], ssem[p], add=True)

            @pl.when((j >= 1) & (j + 3 < NCH))
            def _drain(q=q, j=j):
                pltpu.make_async_copy(bufs[q], acc.at[dst_v.at[j]],
                                      ssem[q]).wait()

            @pl.when(j + 3 < NCH)
            def _prefetch(q=q, j=j):
                pltpu.async_copy(y_hbm.at[src_v.at[j + 3]], bufs[q], gsem[q])
        return carry

    lax.fori_loop(0, NCH // 4, body, 0)
    for p in range(4):
        pltpu.make_async_copy(bufs[p], acc.at[dst_v.at[0]], ssem[p]).wait()
    plsc.subcore_barrier()
    pltpu.sync_copy(acc.at[pl.ds(s * RPT, RPT)],
                    out_hbm.at[c, pl.ds(s * RPT, RPT)])

    @pl.when(s == NS - 1)
    def _tail_out():
        pltpu.sync_copy(acc.at[pl.ds(TAIL_OFF, TAIL)],
                        out_hbm.at[c, pl.ds(TAIL_OFF, TAIL)])


def _prep_body(x_ref, w_ref, b_ref, y1_ref, s1_ref):
    out = jnp.dot(x_ref[...], w_ref[...],
                  preferred_element_type=jnp.float32) + b_ref[...]
    y1_ref[...] = out[:, :H]
    s1_ref[...] = out[:, H:]


def _prep(x, wcat, bcat):
    return pl.pallas_call(
        _prep_body,
        grid=(NB,),
        in_specs=[
            pl.BlockSpec((BLK, D_IN), lambda i: (i, 0)),
            pl.BlockSpec((D_IN, 2 * H), lambda i: (0, 0)),
            pl.BlockSpec((1, 2 * H), lambda i: (0, 0)),
        ],
        out_specs=[
            pl.BlockSpec((BLK, H), lambda i: (i, 0)),
            pl.BlockSpec((BLK, H), lambda i: (i, 0)),
        ],
        out_shape=[
            jax.ShapeDtypeStruct((N, H), jnp.float32),
            jax.ShapeDtypeStruct((N, H), jnp.float32),
        ],
    )(x, wcat, bcat)


def _update_body(gp_ref, eap_ref, we_ref, s_ref, w_ref, b_ref, h_ref, y_ref):
    ea = eap_ref[0] + eap_ref[1]
    cterm = jnp.dot(ea, we_ref[...], preferred_element_type=jnp.float32)
    h = jnp.maximum(gp_ref[0] + gp_ref[1] + cterm + s_ref[...], 0.0)
    h_ref[...] = h
    y_ref[...] = jnp.dot(h, w_ref[...],
                         preferred_element_type=jnp.float32) + b_ref[...]


def _update(gp, eap, we, sterm, wnext, bnext):
    return pl.pallas_call(
        _update_body,
        grid=(NB,),
        in_specs=[
            pl.BlockSpec((NC, BLK, H), lambda i: (0, i, 0)),
            pl.BlockSpec((NC, BLK, D_EDGE), lambda i: (0, i, 0)),
            pl.BlockSpec((D_EDGE, H), lambda i: (0, 0)),
            pl.BlockSpec((BLK, H), lambda i: (i, 0)),
            pl.BlockSpec((H, H), lambda i: (0, 0)),
            pl.BlockSpec((1, H), lambda i: (0, 0)),
        ],
        out_specs=[
            pl.BlockSpec((BLK, H), lambda i: (i, 0)),
            pl.BlockSpec((BLK, H), lambda i: (i, 0)),
        ],
        out_shape=[
            jax.ShapeDtypeStruct((N, H), jnp.float32),
            jax.ShapeDtypeStruct((N, H), jnp.float32),
        ],
    )(gp, eap, we, sterm, wnext, bnext)


def _final_body(gp_ref, eap_ref, we_ref, s_ref, batch_ref, lig_ref, poc_ref,
                wl_ref, bl_ref, wp_ref, bp_ref, wf_ref, bf_ref, wo_ref, bo_ref,
                out_ref, acc_ref):
    i = pl.program_id(0)

    @pl.when(i == 0)
    def _init():
        acc_ref[...] = jnp.zeros_like(acc_ref)

    ea = eap_ref[0] + eap_ref[1]
    cterm = jnp.dot(ea, we_ref[...], preferred_element_type=jnp.float32)
    h3 = jnp.maximum(gp_ref[0] + gp_ref[1] + cterm + s_ref[...], 0.0)

    ids = batch_ref[0, 0, :]
    onehot = (ids[None, :] == lax.broadcasted_iota(jnp.int32, (B, BLK), 0)
              ).astype(jnp.float32)
    hcat = jnp.concatenate(
        [h3, jnp.ones((BLK, 1), jnp.float32),
         jnp.zeros((BLK, 2 * H - H - 1), jnp.float32)], axis=1)
    acc_ref[...] += jnp.dot(onehot, hcat, preferred_element_type=jnp.float32)

    @pl.when(i == NB - 1)
    def _finish():
        sums = acc_ref[:, :H]
        counts = acc_ref[:, H:H + 1]
        pooled = sums / jnp.maximum(counts, 1.0)
        lig = jnp.dot(lig_ref[...], wl_ref[...],
                      preferred_element_type=jnp.float32) + bl_ref[...]
        poc = jnp.dot(poc_ref[...], wp_ref[...],
                      preferred_element_type=jnp.float32) + bp_ref[...]
        zcat = jnp.concatenate([pooled, lig, poc], axis=1)
        z = jnp.dot(zcat, wf_ref[...],
                    preferred_element_type=jnp.float32) + bf_ref[...]
        out_ref[...] = jnp.dot(z, wo_ref[...],
                               preferred_element_type=jnp.float32) + bo_ref[...]


def _final(gp, eap, we, sterm, batch3, lig, poc, wl, bl, wp, bp, wf, bf, wo, bo):
    return pl.pallas_call(
        _final_body,
        grid=(NB,),
        in_specs=[
            pl.BlockSpec((NC, BLK, H), lambda i: (0, i, 0)),
            pl.BlockSpec((NC, BLK, D_EDGE), lambda i: (0, i, 0)),
            pl.BlockSpec((D_EDGE, H), lambda i: (0, 0)),
            pl.BlockSpec((BLK, H), lambda i: (i, 0)),
            pl.BlockSpec((1, 1, BLK), lambda i: (i, 0, 0)),
            pl.BlockSpec((B, D_IN), lambda i: (0, 0)),
            pl.BlockSpec((B, D_IN), lambda i: (0, 0)),
            pl.BlockSpec((D_IN, H), lambda i: (0, 0)),
            pl.BlockSpec((1, H), lambda i: (0, 0)),
            pl.BlockSpec((D_IN, H), lambda i: (0, 0)),
            pl.BlockSpec((1, H), lambda i: (0, 0)),
            pl.BlockSpec((3 * H, H), lambda i: (0, 0)),
            pl.BlockSpec((1, H), lambda i: (0, 0)),
            pl.BlockSpec((H, 1), lambda i: (0, 0)),
            pl.BlockSpec((1, 1), lambda i: (0, 0)),
        ],
        out_specs=pl.BlockSpec((B, 1), lambda i: (0, 0)),
        out_shape=jax.ShapeDtypeStruct((B, 1), jnp.float32),
        scratch_shapes=[pltpu.VMEM((B, 2 * H), jnp.float32)],
    )(gp, eap, we, sterm, batch3, lig, poc, wl, bl, wp, bp, wf, bf, wo, bo)


def kernel(x, edge_index, edge_attr, batch, ligand_features, pocket_features,
           return_embeddings, W_msg1, b_msg1, W_edge1, b_edge1, W_self1,
           b_self1, W_msg2, b_msg2, W_edge2, b_edge2, W_msg3, b_msg3, W_edge3,
           b_edge3, W_lig, b_lig, W_poc, b_poc, W_fus, b_fus, W_out, b_out):
    src_r = edge_index[0].reshape(NW, NCH, CH)
    dst_r = edge_index[1].reshape(NW, NCH, CH)
    ea_r = edge_attr.reshape(NW, NCH, CH, D_EDGE)
    batch3 = batch.reshape(NB, 1, BLK)
    z16 = jnp.zeros((RPT, D_EDGE), jnp.float32)
    z64 = jnp.zeros((RPT, H), jnp.float32)

    eap = _ea_kernel(ea_r, dst_r, z16)

    wcat = jnp.concatenate([W_msg1, W_self1], axis=1)
    bcat = jnp.concatenate([b_msg1 + b_edge1, b_self1]).reshape(1, 2 * H)
    y1, s1 = _prep(x, wcat, bcat)

    g1 = _gather_scatter_kernel(y1, src_r, dst_r, z64)
    h1, y2 = _update(g1, eap, W_edge1, s1, W_msg2,
                     (b_msg2 + b_edge2).reshape(1, H))
    g2 = _gather_scatter_kernel(y2, src_r, dst_r, z64)
    h2, y3 = _update(g2, eap, W_edge2, h1, W_msg3,
                     (b_msg3 + b_edge3).reshape(1, H))
    g3 = _gather_scatter_kernel(y3, src_r, dst_r, z64)
    out = _final(g3, eap, W_edge3, h2, batch3, ligand_features,
                 pocket_features, W_lig, b_lig.reshape(1, H), W_poc,
                 b_poc.reshape(1, H), W_fus, b_fus.reshape(1, H), W_out,
                 b_out.reshape(1, 1))

    gate = jnp.asarray(return_embeddings, jnp.float32)
    return out * (1.0 - gate)
